# Initial kernel scaffold; baseline (speedup 1.0000x reference)
import jax
import jax.numpy as jnp
from jax.experimental import pallas as pl


def kernel(x, edge_index, edge_values, theta, bias):
    raise NotImplementedError("write your pallas kernel here")


# speedup vs baseline: 3.6879x; 3.6879x over previous
import functools

import jax
import jax.numpy as jnp
from jax import lax
from jax.experimental import pallas as pl
from jax.experimental.pallas import tpu as pltpu
from jax.experimental.pallas import tpu_sc as plsc

_NC = 2
_NS = 16
_NW = _NC * _NS
_CHUNK = 64
_LANES = 16


@functools.lru_cache(maxsize=None)
def _make_spmm(M, M_pad, CH, nchunk):
    rows_per_tile = M_pad // _NS

    mesh = plsc.VectorSubcoreMesh(core_axis_name="c", subcore_axis_name="s")

    @functools.partial(
        pl.kernel,
        out_type=jax.ShapeDtypeStruct((_NC, 2, M_pad, CH), jnp.float32),
        mesh=mesh,
        scratch_types=[
            pltpu.VMEM((nchunk, _CHUNK), jnp.int32),
            pltpu.VMEM((nchunk, _CHUNK), jnp.int32),
            pltpu.VMEM((nchunk, _CHUNK), jnp.float32),
            pltpu.VMEM((_CHUNK, CH), jnp.float32),
            pltpu.VMEM_SHARED((M_pad, CH), jnp.float32),
            pltpu.SemaphoreType.DMA,
        ],
        compiler_params=pltpu.CompilerParams(use_tc_tiling_on_sc=False),
    )
    def spmm(x0_hbm, x1_hbm, rows_hbm, cols_hbm, vals_hbm, zinit_hbm,
             out_hbm, rows_v, cols_v, vals_v, gbuf, yacc, sem):
        c = lax.axis_index("c")
        s = lax.axis_index("s")
        wid = c * _NS + s

        pltpu.sync_copy(rows_hbm.at[wid], rows_v)
        pltpu.sync_copy(cols_hbm.at[wid], cols_v)
        pltpu.sync_copy(vals_hbm.at[wid], vals_v)
        base = pl.multiple_of(s * rows_per_tile, 8)

        for h, x_hbm in enumerate((x0_hbm, x1_hbm)):
            pltpu.sync_copy(zinit_hbm, yacc.at[pl.ds(base, rows_per_tile)])
            plsc.subcore_barrier()

            def chunk_body(j, carry):
                pltpu.async_copy(x_hbm.at[cols_v.at[j]], gbuf, sem).wait()
                for eg in range(_CHUNK // _LANES):
                    v16 = vals_v[j, pl.ds(eg * _LANES, _LANES)]
                    for l in range(_LANES):
                        e = eg * _LANES + l
                        v = jnp.broadcast_to(v16[l], (_LANES,))
                        for g in range(CH // _LANES):
                            sl = pl.ds(g * _LANES, _LANES)
                            gbuf[e, sl] = gbuf[e, sl] * v
                pltpu.sync_copy(gbuf, yacc.at[rows_v.at[j]], add=True)
                return carry

            lax.fori_loop(0, nchunk, chunk_body, 0)
            plsc.subcore_barrier()
            pltpu.sync_copy(
                yacc.at[pl.ds(base, rows_per_tile)],
                out_hbm.at[c, h, pl.ds(base, rows_per_tile)])

    return spmm


def _combine(a, b):
    def body(a_ref, b_ref, o_ref):
        o_ref[...] = a_ref[...] + b_ref[...]

    return pl.pallas_call(
        body, out_shape=jax.ShapeDtypeStruct(a.shape, a.dtype))(a, b)


def _dense(M, y1_halves, p20, p21, w0, w1, bias_col):
    C_out = w0.shape[0]
    CH = y1_halves.shape[2]

    def body(y1_ref, pa_ref, pb_ref, w0_ref, w1_ref, b_ref, o_ref):
        dn = (((1,), (1,)), ((), ()))
        acc = lax.dot_general(w0_ref[0, :, :CH], y1_ref[0],
                              dn, preferred_element_type=jnp.float32)
        acc += lax.dot_general(w0_ref[0, :, CH:], y1_ref[1],
                               dn, preferred_element_type=jnp.float32)
        y2_lo = pa_ref[0] + pb_ref[0]
        y2_hi = pa_ref[1] + pb_ref[1]
        acc += lax.dot_general(w1_ref[0, :, :CH], y2_lo,
                               dn, preferred_element_type=jnp.float32)
        acc += lax.dot_general(w1_ref[0, :, CH:], y2_hi,
                               dn, preferred_element_type=jnp.float32)
        o_ref[0] = acc[:, :M] + b_ref[...]

    return pl.pallas_call(
        body, out_shape=jax.ShapeDtypeStruct((1, C_out, M), jnp.float32),
    )(y1_halves, p20, p21, w0[None], w1[None], bias_col)


def kernel(x, edge_index, edge_values, theta, bias):
    _, C_in, M = x.shape
    E = edge_index.shape[1]
    per = _NW * _CHUNK
    nchunk = -(-E // per)
    E_pad = nchunk * per

    rows = edge_index[0]
    cols = edge_index[1]
    vals = edge_values
    if E_pad != E:
        rows = jnp.pad(rows, (0, E_pad - E))
        cols = jnp.pad(cols, (0, E_pad - E))
        vals = jnp.pad(vals, (0, E_pad - E))
    rows3 = rows.reshape(_NW, nchunk, _CHUNK)
    cols3 = cols.reshape(_NW, nchunk, _CHUNK)
    vals3 = vals.reshape(_NW, nchunk, _CHUNK)

    rpt = (-(-M // _NS) + 7) // 8 * 8
    M_pad = rpt * _NS
    CH = C_in // 2

    x_mc = x[0].T
    x0 = x_mc[:, :CH]
    x1 = x_mc[:, CH:]
    zinit = jnp.zeros((rpt, CH), jnp.float32)

    spmm = _make_spmm(M, M_pad, CH, nchunk)
    p1 = spmm(x0, x1, rows3, cols3, vals3, zinit)
    y1h = _combine(p1[0], p1[1])
    p2 = spmm(y1h[0], y1h[1], rows3, cols3, vals3, zinit)
    out = _dense(M, y1h, p2[0], p2[1], theta[:, :, 0], theta[:, :, 1],
                 bias[0])
    return out
